# baseline (speedup 1.0000x reference)
import functools

import jax
import jax.numpy as jnp
from jax import lax
from jax.experimental import pallas as pl
from jax.experimental.pallas import tpu as pltpu
from jax.experimental.pallas import tpu_sc as plsc

B = 4096
NC_FEAT = 13
F = 26
V = 100000
D = 64
H1, H2, H3 = 512, 256, 128

R = B * F
NW = 32
RPW = R // NW
CH = 832
NCHUNK = RPW // CH


def _gather_body(tab_hbm, idx_hbm, out_hbm, idx_v, buf0, buf1, sem0, sem1):
    wid = lax.axis_index("s") * 2 + lax.axis_index("c")
    base = wid * RPW
    pltpu.sync_copy(idx_hbm.at[pl.ds(base, RPW)], idx_v)
    bufs = (buf0, buf1)
    sems = (sem0, sem1)
    pltpu.async_copy(tab_hbm.at[idx_v.at[pl.ds(0, CH)]], bufs[0], sems[0])
    for ci in range(1, NCHUNK):
        pltpu.async_copy(
            tab_hbm.at[idx_v.at[pl.ds(ci * CH, CH)]], bufs[ci % 2], sems[ci % 2]
        )
        pltpu.make_async_copy(
            tab_hbm.at[idx_v.at[pl.ds((ci - 1) * CH, CH)]],
            bufs[(ci - 1) % 2],
            sems[(ci - 1) % 2],
        ).wait()
        pltpu.sync_copy(
            bufs[(ci - 1) % 2], out_hbm.at[pl.ds(base + (ci - 1) * CH, CH)]
        )
    last = NCHUNK - 1
    pltpu.make_async_copy(
        tab_hbm.at[idx_v.at[pl.ds(last * CH, CH)]], bufs[last % 2], sems[last % 2]
    ).wait()
    pltpu.sync_copy(bufs[last % 2], out_hbm.at[pl.ds(base + last * CH, CH)])


def _sc_gather(flat_tables, flat_idx):
    mesh = plsc.VectorSubcoreMesh(core_axis_name="c", subcore_axis_name="s")
    k = functools.partial(
        pl.kernel,
        mesh=mesh,
        out_type=jax.ShapeDtypeStruct((R, D), jnp.float32),
        scratch_types=[
            pltpu.VMEM((RPW,), jnp.int32),
            pltpu.VMEM((CH, D), jnp.float32),
            pltpu.VMEM((CH, D), jnp.float32),
            pltpu.SemaphoreType.DMA,
            pltpu.SemaphoreType.DMA,
        ],
        compiler_params=pltpu.CompilerParams(use_tc_tiling_on_sc=False),
    )(_gather_body)
    return k(flat_tables, flat_idx)


def _mlp_body(cont_ref, emb_ref, wc_ref, bc_ref, w1a_ref, w1b_ref, b1_ref,
              w2_ref, b2_ref, w3_ref, b3_ref, wo_ref, bo_ref, out_ref):
    f32 = jnp.float32
    x = jnp.dot(cont_ref[...], wc_ref[...], preferred_element_type=f32) + bc_ref[...]
    h = (jnp.dot(x, w1a_ref[...], preferred_element_type=f32)
         + jnp.dot(emb_ref[...], w1b_ref[...], preferred_element_type=f32)
         + b1_ref[...])
    h = jnp.maximum(h, 0.0)
    h = jnp.maximum(jnp.dot(h, w2_ref[...], preferred_element_type=f32) + b2_ref[...], 0.0)
    h = jnp.maximum(jnp.dot(h, w3_ref[...], preferred_element_type=f32) + b3_ref[...], 0.0)
    logit = jnp.dot(h, wo_ref[...], preferred_element_type=f32) + bo_ref[...]
    out_ref[...] = jax.nn.sigmoid(logit)


def _tc_mlp(cont, emb2d, W_cont, b_cont, W1a, W1b, b1, W2, b2, W3, b3, Wo, bo):
    BB = 512
    grid = (B // BB,)
    full = lambda a: pl.BlockSpec(a.shape, lambda i: (0,) * a.ndim)
    return pl.pallas_call(
        _mlp_body,
        grid=grid,
        in_specs=[
            pl.BlockSpec((BB, NC_FEAT), lambda i: (i, 0)),
            pl.BlockSpec((BB, F * D), lambda i: (i, 0)),
            full(W_cont), full(b_cont), full(W1a), full(W1b), full(b1),
            full(W2), full(b2), full(W3), full(b3), full(Wo), full(bo),
        ],
        out_specs=pl.BlockSpec((BB, 1), lambda i: (i, 0)),
        out_shape=jax.ShapeDtypeStruct((B, 1), jnp.float32),
    )(cont, emb2d, W_cont, b_cont, W1a, W1b, b1, W2, b2, W3, b3, Wo, bo)


def kernel(continuous_features, categorical_features, W_cont, b_cont, tables,
           W1, b1, W2, b2, W3, b3, Wo, bo):
    flat_tables = tables.reshape(F * V, D)
    offs = (jnp.arange(F, dtype=jnp.int32) * V)[None, :]
    flat_idx = (categorical_features.astype(jnp.int32) + offs).reshape(-1)
    emb = _sc_gather(flat_tables, flat_idx)
    emb2d = emb.reshape(B, F * D)
    W1a = W1[:D]
    W1b = W1[D:]
    return _tc_mlp(
        continuous_features, emb2d,
        W_cont, b_cont.reshape(1, D),
        W1a, W1b, b1.reshape(1, H1),
        W2, b2.reshape(1, H2),
        W3, b3.reshape(1, H3),
        Wo, bo.reshape(1, 1),
    )


# speedup vs baseline: 1.6565x; 1.6565x over previous
import functools

import jax
import jax.numpy as jnp
from jax import lax
from jax.experimental import pallas as pl
from jax.experimental.pallas import tpu as pltpu
from jax.experimental.pallas import tpu_sc as plsc

B = 4096
NC_FEAT = 13
F = 26
V = 100000
D = 64
H1, H2, H3 = 512, 256, 128

FPC = F // 2
NSLAB = FPC * 8
L0 = 49920
L1 = 50048
LT = 50176
NVREG = B // 16


def _gather_pass(idx_v, row_v, out_v, vh):
    zero = jnp.zeros((16,), jnp.float32)

    def body(i, _):
        iv = idx_v[pl.ds(i * 16, 16)]
        if vh == 0:
            m = iv < L0
            ic = jnp.clip(iv, 0, L0 - 1)
        else:
            m = iv >= L0
            ic = jnp.clip(iv - L0, 0, LT - 1)
        g = plsc.load_gather(row_v, [ic])
        out_v[pl.ds(i * 16, 16)] = jnp.where(m, g, zero)
        return 0

    lax.fori_loop(0, NVREG, body, 0)


def _emb_body(tab_hbm, tail_hbm, idx_hbm, out_hbm,
              idx_v, row_v, out_v,
              stage0, stage1, tslab0, tslab1, ostage,
              sem_a, sem_b, sem_ta, sem_tb):
    c = lax.axis_index("c")
    tid = lax.axis_index("s")
    dd = tid & 7
    vh = tid >> 3
    stages = (stage0, stage1)
    ssems = (sem_a, sem_b)
    tslabs = (tslab0, tslab1)
    tsems = (sem_ta, sem_tb)

    def row0(s):
        return pl.multiple_of((c * FPC + (s >> 3)) * D + (s & 7) * 8, 8)

    def slab_src(s, h):
        if h == 0:
            return tab_hbm.at[pl.ds(row0(s), 8), pl.ds(0, L0)]
        return tab_hbm.at[pl.ds(row0(s), 8), pl.ds(L0, L1)]

    def tail_src(s):
        return tail_hbm.at[pl.ds(row0(s), 8), :]

    @pl.when(tid == 0)
    def _():
        pltpu.async_copy(slab_src(0, 0), stage0, sem_a)
        pltpu.async_copy(tail_src(0), tslab0, sem_ta)

    def step(s, h, tb):
        @pl.when(tid == 0)
        def _():
            pltpu.make_async_copy(slab_src(s, h), stages[h], ssems[h]).wait()
            if h == 1:
                pltpu.make_async_copy(tail_src(s), tslabs[tb], tsems[tb]).wait()

        plsc.subcore_barrier()

        @pl.when(tid == 0)
        def _():
            if h == 0:
                pltpu.async_copy(slab_src(s, 1), stages[1], ssems[1])
            else:
                @pl.when(s + 1 < NSLAB)
                def _():
                    pltpu.async_copy(slab_src(s + 1, 0), stages[0], ssems[0])
                    pltpu.async_copy(tail_src(s + 1), tslabs[tb ^ 1],
                                     tsems[tb ^ 1])

        @pl.when(((s & 7) == 0) & (h == 0))
        def _():
            off = pl.multiple_of((c * FPC + (s >> 3)) * B, 8)
            pltpu.sync_copy(idx_hbm.at[pl.ds(off, B)], idx_v)

        @pl.when(vh == h)
        def _():
            if h == 0:
                pltpu.sync_copy(stages[0].at[dd], row_v.at[pl.ds(0, L0)])
            else:
                pltpu.sync_copy(stages[1].at[dd], row_v.at[pl.ds(0, L1)])
                pltpu.sync_copy(tslabs[tb].at[dd], row_v.at[pl.ds(L1, 128)])
            _gather_pass(idx_v, row_v, out_v, h)
            pltpu.sync_copy(out_v, ostage.at[dd * 2 + h])

        if h == 1:
            plsc.subcore_barrier()

            @pl.when(tid == 0)
            def _():
                pltpu.sync_copy(ostage, out_hbm.at[c * NSLAB + s])

    def outer(p, _):
        step(2 * p, 0, 0)
        step(2 * p, 1, 0)
        step(2 * p + 1, 0, 1)
        step(2 * p + 1, 1, 1)
        return 0

    lax.fori_loop(0, NSLAB // 2, outer, 0)


def _sc_embed_t(tab2d, tail128, idx_flat):
    mesh = plsc.VectorSubcoreMesh(core_axis_name="c", subcore_axis_name="s")
    k = functools.partial(
        pl.kernel,
        mesh=mesh,
        out_type=jax.ShapeDtypeStruct((2 * NSLAB, 16, B), jnp.float32),
        scratch_types=[
            pltpu.VMEM((B,), jnp.int32),
            pltpu.VMEM((LT,), jnp.float32),
            pltpu.VMEM((B,), jnp.float32),
            pltpu.VMEM_SHARED((8, L0), jnp.float32),
            pltpu.VMEM_SHARED((8, L1), jnp.float32),
            pltpu.VMEM_SHARED((8, 128), jnp.float32),
            pltpu.VMEM_SHARED((8, 128), jnp.float32),
            pltpu.VMEM_SHARED((16, B), jnp.float32),
            pltpu.SemaphoreType.DMA,
            pltpu.SemaphoreType.DMA,
            pltpu.SemaphoreType.DMA,
            pltpu.SemaphoreType.DMA,
        ],
        compiler_params=pltpu.CompilerParams(needs_layout_passes=False),
    )(_emb_body)
    return k(tab2d, tail128, idx_flat)


def _mlp_body(cont_ref, emb_ref, wc_ref, bc_ref, w1a_ref, w1b_ref, b1_ref,
              w2_ref, b2_ref, w3_ref, b3_ref, wo_ref, bo_ref, out_ref):
    f32 = jnp.float32
    dot = lambda a, b: jnp.dot(a, b, preferred_element_type=f32)
    xt = dot(wc_ref[...], cont_ref[...]) + bc_ref[...]
    h = dot(w1a_ref[...], xt) + dot(w1b_ref[...], emb_ref[...]) + b1_ref[...]
    h = jnp.maximum(h, 0.0)
    h = jnp.maximum(dot(w2_ref[...], h) + b2_ref[...], 0.0)
    h = jnp.maximum(dot(w3_ref[...], h) + b3_ref[...], 0.0)
    logit = dot(wo_ref[...], h) + bo_ref[...]
    out_ref[...] = jax.nn.sigmoid(logit)


def _tc_mlp_t(cont_t, emb_t, WcT, bcC, W1aT, W1bT, b1C, W2T, b2C, W3T, b3C,
              WoT, boC):
    BB = 512
    grid = (B // BB,)
    full = lambda a: pl.BlockSpec(a.shape, lambda i: (0,) * a.ndim)
    return pl.pallas_call(
        _mlp_body,
        grid=grid,
        in_specs=[
            pl.BlockSpec((NC_FEAT, BB), lambda i: (0, i)),
            pl.BlockSpec((2 * F * D, BB), lambda i: (0, i)),
            full(WcT), full(bcC), full(W1aT), full(W1bT), full(b1C),
            full(W2T), full(b2C), full(W3T), full(b3C), full(WoT), full(boC),
        ],
        out_specs=pl.BlockSpec((1, BB), lambda i: (0, i)),
        out_shape=jax.ShapeDtypeStruct((1, B), jnp.float32),
    )(cont_t, emb_t, WcT, bcC, W1aT, W1bT, b1C, W2T, b2C, W3T, b3C, WoT, boC)


def kernel(continuous_features, categorical_features, W_cont, b_cont, tables,
           W1, b1, W2, b2, W3, b3, Wo, bo):
    tab2d = tables.transpose(0, 2, 1).reshape(F * D, V)
    tail128 = jnp.pad(lax.slice(tab2d, (0, 99968), (F * D, V)),
                      ((0, 0), (0, 96)))
    idx_flat = categorical_features.astype(jnp.int32).T.reshape(F * B)
    emb3 = _sc_embed_t(tab2d, tail128, idx_flat)
    emb2 = emb3.reshape(2 * F * D, B)

    W1T = W1.T
    W1bT2 = jnp.repeat(W1T[:, D:], 2, axis=1)
    out_t = _tc_mlp_t(
        continuous_features.T, emb2,
        W_cont.T, b_cont.reshape(D, 1),
        W1T[:, :D], W1bT2, b1.reshape(H1, 1),
        W2.T, b2.reshape(H2, 1),
        W3.T, b3.reshape(H3, 1),
        Wo.T, bo.reshape(1, 1),
    )
    return out_t.reshape(B, 1)
